# baseline (speedup 1.0000x reference)
import functools

import jax
import jax.numpy as jnp
from jax.experimental import pallas as pl
from jax.experimental.pallas import tpu as pltpu

_BN = 2048
_BK = 4096
_LANES = 128


def _dist_argmin_kernel(x_ref, cb_ref, idx_ref, dist_ref, rmin_sc, rarg_sc,
                        c2_sc, x2_sc):
    i = pl.program_id(0)
    k = pl.program_id(1)
    nk = pl.num_programs(1)

    x = x_ref[...]
    cb = cb_ref[...]
    bk = cb.shape[0]

    @pl.when(k == 0)
    def _init():
        rmin_sc[...] = jnp.full(rmin_sc.shape, jnp.inf, jnp.float32)
        rarg_sc[...] = jnp.zeros(rarg_sc.shape, jnp.int32)
        x2_sc[...] = jnp.sum(x * x, axis=1, keepdims=True)

    @pl.when(i == 0)
    def _cache_c2():
        c2_col = jnp.sum(cb * cb, axis=1, keepdims=True)
        c2_sc[:, pl.ds(k * bk, bk)] = c2_col.T

    m2dot = jax.lax.dot_general(
        x * jnp.float32(-2.0), cb, (((1,), (1,)), ((), ())),
        preferred_element_type=jnp.float32)
    c2 = c2_sc[:, pl.ds(k * bk, bk)]
    x2 = x2_sc[...]

    lanes = _LANES
    lane_iota = jax.lax.broadcasted_iota(jnp.int32, (1, lanes), 1)
    rmin = rmin_sc[...]
    rarg = rarg_sc[...]
    for g in range(bk // lanes):
        sl = slice(g * lanes, (g + 1) * lanes)
        d2 = (x2 + c2[:, sl]) + m2dot[:, sl]
        cand = lane_iota + (k * bk + g * lanes)
        lt = d2 < rmin
        rarg = jnp.where(lt, cand, rarg)
        rmin = jnp.minimum(rmin, d2)
    rmin_sc[...] = rmin
    rarg_sc[...] = rarg

    @pl.when(k == nk - 1)
    def _write():
        dist = jnp.sqrt(jnp.maximum(rmin, 0.0))
        m = jnp.min(dist, axis=1, keepdims=True)
        cand = jnp.where(dist == m, rarg, jnp.iinfo(jnp.int32).max)
        idx_ref[...] = jnp.min(cand, axis=1, keepdims=True)
        dist_ref[...] = m


@functools.partial(jax.jit, static_argnames=())
def _assign(X, cb):
    n, d = X.shape
    kk = cb.shape[0]
    bn, bk = _BN, _BK
    grid = (n // bn, kk // bk)
    idx2, dist2 = pl.pallas_call(
        _dist_argmin_kernel,
        grid=grid,
        in_specs=[
            pl.BlockSpec((bn, d), lambda i, k: (i, 0)),
            pl.BlockSpec((bk, d), lambda i, k: (k, 0)),
        ],
        out_specs=[
            pl.BlockSpec((bn, 1), lambda i, k: (i, 0)),
            pl.BlockSpec((bn, 1), lambda i, k: (i, 0)),
        ],
        out_shape=[
            jax.ShapeDtypeStruct((n, 1), jnp.int32),
            jax.ShapeDtypeStruct((n, 1), jnp.float32),
        ],
        scratch_shapes=[
            pltpu.VMEM((bn, _LANES), jnp.float32),
            pltpu.VMEM((bn, _LANES), jnp.int32),
            pltpu.VMEM((1, kk), jnp.float32),
            pltpu.VMEM((bn, 1), jnp.float32),
        ],
        compiler_params=pltpu.CompilerParams(
            dimension_semantics=("parallel", "arbitrary"),
        ),
    )(X, cb)
    return idx2[:, 0], dist2[:, 0]


def kernel(X, codebook, return_dist):
    idx, dist = _assign(X, codebook)
    dist = dist * jnp.asarray(return_dist, dist.dtype)
    return (idx, dist)


# speedup vs baseline: 1.0280x; 1.0280x over previous
import functools

import jax
import jax.numpy as jnp
from jax.experimental import pallas as pl
from jax.experimental.pallas import tpu as pltpu

_BN = 4096
_BK = 2048
_LANES = 128


def _dist_argmin_kernel(x_ref, cb_ref, idx_ref, dist_ref, rmin_sc, rarg_sc,
                        c2_sc, x2_sc):
    i = pl.program_id(0)
    k = pl.program_id(1)
    nk = pl.num_programs(1)

    x = x_ref[...]
    cb = cb_ref[...]
    bk = cb.shape[0]

    @pl.when(k == 0)
    def _init():
        rmin_sc[...] = jnp.full(rmin_sc.shape, jnp.inf, jnp.float32)
        rarg_sc[...] = jnp.zeros(rarg_sc.shape, jnp.int32)
        x2_sc[...] = jnp.sum(x * x, axis=1, keepdims=True)

    @pl.when(i == 0)
    def _cache_c2():
        c2_col = jnp.sum(cb * cb, axis=1, keepdims=True)
        c2_sc[:, pl.ds(k * bk, bk)] = c2_col.T

    m2dot = jax.lax.dot_general(
        x * jnp.float32(-2.0), cb, (((1,), (1,)), ((), ())),
        preferred_element_type=jnp.float32)
    c2 = c2_sc[:, pl.ds(k * bk, bk)]
    x2 = x2_sc[...]

    lanes = _LANES
    lane_iota = jax.lax.broadcasted_iota(jnp.int32, (1, lanes), 1)
    rmin = rmin_sc[...]
    rarg = rarg_sc[...]
    for g in range(bk // lanes):
        sl = slice(g * lanes, (g + 1) * lanes)
        d2 = (x2 + c2[:, sl]) + m2dot[:, sl]
        cand = lane_iota + (k * bk + g * lanes)
        lt = d2 < rmin
        rarg = jnp.where(lt, cand, rarg)
        rmin = jnp.minimum(rmin, d2)
    rmin_sc[...] = rmin
    rarg_sc[...] = rarg

    @pl.when(k == nk - 1)
    def _write():
        dist = jnp.sqrt(jnp.maximum(rmin, 0.0))
        m = jnp.min(dist, axis=1, keepdims=True)
        cand = jnp.where(dist == m, rarg, jnp.iinfo(jnp.int32).max)
        idx_ref[...] = jnp.min(cand, axis=1, keepdims=True)
        dist_ref[...] = m


@functools.partial(jax.jit, static_argnames=())
def _assign(X, cb):
    n, d = X.shape
    kk = cb.shape[0]
    bn, bk = _BN, _BK
    grid = (n // bn, kk // bk)
    idx2, dist2 = pl.pallas_call(
        _dist_argmin_kernel,
        grid=grid,
        in_specs=[
            pl.BlockSpec((bn, d), lambda i, k: (i, 0)),
            pl.BlockSpec((bk, d), lambda i, k: (k, 0)),
        ],
        out_specs=[
            pl.BlockSpec((bn, 1), lambda i, k: (i, 0)),
            pl.BlockSpec((bn, 1), lambda i, k: (i, 0)),
        ],
        out_shape=[
            jax.ShapeDtypeStruct((n, 1), jnp.int32),
            jax.ShapeDtypeStruct((n, 1), jnp.float32),
        ],
        scratch_shapes=[
            pltpu.VMEM((bn, _LANES), jnp.float32),
            pltpu.VMEM((bn, _LANES), jnp.int32),
            pltpu.VMEM((1, kk), jnp.float32),
            pltpu.VMEM((bn, 1), jnp.float32),
        ],
        compiler_params=pltpu.CompilerParams(
            dimension_semantics=("parallel", "arbitrary"),
        ),
    )(X, cb)
    return idx2[:, 0], dist2[:, 0]


def kernel(X, codebook, return_dist):
    idx, dist = _assign(X, codebook)
    dist = dist * jnp.asarray(return_dist, dist.dtype)
    return (idx, dist)


# speedup vs baseline: 1.0843x; 1.0548x over previous
import functools

import jax
import jax.numpy as jnp
from jax.experimental import pallas as pl
from jax.experimental.pallas import tpu as pltpu

_BN = 4096
_BK = 2048
_LANES = 128


def _dist_argmin_kernel(x_ref, cb_ref, idx_ref, dist_ref, rmin_sc, rarg_sc,
                        c2_sc, x2_sc, xm2_sc):
    i = pl.program_id(0)
    k = pl.program_id(1)
    nk = pl.num_programs(1)

    cb = cb_ref[...]
    bk = cb.shape[0]

    @pl.when(k == 0)
    def _init():
        x = x_ref[...]
        rmin_sc[...] = jnp.full(rmin_sc.shape, jnp.inf, jnp.float32)
        rarg_sc[...] = jnp.zeros(rarg_sc.shape, jnp.int32)
        x2_sc[...] = jnp.sum(x * x, axis=1, keepdims=True)
        xm2_sc[...] = x * jnp.float32(-2.0)

    @pl.when(i == 0)
    def _cache_c2():
        c2_col = jnp.sum(cb * cb, axis=1, keepdims=True)
        c2_sc[:, pl.ds(k * bk, bk)] = c2_col.T

    m2dot = jax.lax.dot_general(
        xm2_sc[...], cb, (((1,), (1,)), ((), ())),
        preferred_element_type=jnp.float32)
    c2 = c2_sc[:, pl.ds(k * bk, bk)]
    x2 = x2_sc[...]

    lanes = _LANES
    lane_iota = jax.lax.broadcasted_iota(jnp.int32, (1, lanes), 1)
    rmin = rmin_sc[...]
    rarg = rarg_sc[...]
    for g in range(bk // lanes):
        sl = slice(g * lanes, (g + 1) * lanes)
        d2 = (x2 + c2[:, sl]) + m2dot[:, sl]
        cand = lane_iota + (k * bk + g * lanes)
        lt = d2 < rmin
        rarg = jnp.where(lt, cand, rarg)
        rmin = jnp.minimum(rmin, d2)
    rmin_sc[...] = rmin
    rarg_sc[...] = rarg

    @pl.when(k == nk - 1)
    def _write():
        dist = jnp.sqrt(jnp.maximum(rmin, 0.0))
        m = jnp.min(dist, axis=1, keepdims=True)
        cand = jnp.where(dist == m, rarg, jnp.iinfo(jnp.int32).max)
        idx_ref[...] = jnp.min(cand, axis=1, keepdims=True)
        dist_ref[...] = m


@functools.partial(jax.jit, static_argnames=())
def _assign(X, cb):
    n, d = X.shape
    kk = cb.shape[0]
    bn, bk = _BN, _BK
    grid = (n // bn, kk // bk)
    idx2, dist2 = pl.pallas_call(
        _dist_argmin_kernel,
        grid=grid,
        in_specs=[
            pl.BlockSpec((bn, d), lambda i, k: (i, 0)),
            pl.BlockSpec((bk, d), lambda i, k: (k, 0)),
        ],
        out_specs=[
            pl.BlockSpec((bn, 1), lambda i, k: (i, 0)),
            pl.BlockSpec((bn, 1), lambda i, k: (i, 0)),
        ],
        out_shape=[
            jax.ShapeDtypeStruct((n, 1), jnp.int32),
            jax.ShapeDtypeStruct((n, 1), jnp.float32),
        ],
        scratch_shapes=[
            pltpu.VMEM((bn, _LANES), jnp.float32),
            pltpu.VMEM((bn, _LANES), jnp.int32),
            pltpu.VMEM((1, kk), jnp.float32),
            pltpu.VMEM((bn, 1), jnp.float32),
            pltpu.VMEM((bn, d), jnp.float32),
        ],
        compiler_params=pltpu.CompilerParams(
            dimension_semantics=("parallel", "arbitrary"),
        ),
    )(X, cb)
    return idx2[:, 0], dist2[:, 0]


def kernel(X, codebook, return_dist):
    idx, dist = _assign(X, codebook)
    dist = dist * jnp.asarray(return_dist, dist.dtype)
    return (idx, dist)
